# Initial kernel scaffold; baseline (speedup 1.0000x reference)
import jax
import jax.numpy as jnp
from jax.experimental import pallas as pl


def kernel(h, edge_index, W0a, b0a, W0b, b0b, W1a, b1a, W1b, b1b):
    raise NotImplementedError("write your pallas kernel here")


# speedup vs baseline: 4.8711x; 4.8711x over previous
import functools

import jax
import jax.numpy as jnp
from jax import lax
from jax.experimental import pallas as pl
from jax.experimental.pallas import tpu as pltpu
from jax.experimental.pallas import tpu_sc as plsc

N = 10000
D = 128
DH = D // 2
E = 320000

NC = 2
NS = 16

CHUNK = 128
CHUNKS = 160
NBUF = 4
E_PAD = NS * CHUNKS * CHUNK

AGG_ROWS = 10240
ZROWS = 128
OUT_PER_TILE = AGG_ROWS // NS


def _sc_aggregate(xs, src3, dst3, zeros_blk):
    mesh = plsc.VectorSubcoreMesh(core_axis_name="c", subcore_axis_name="s")

    @functools.partial(
        pl.kernel,
        out_type=jax.ShapeDtypeStruct((NC, AGG_ROWS, DH), jnp.float32),
        mesh=mesh,
        scratch_types=[
            pltpu.VMEM((CHUNKS, CHUNK), jnp.int32),
            pltpu.VMEM((CHUNKS, CHUNK), jnp.int32),
            pltpu.VMEM((NBUF, CHUNK, DH), jnp.float32),
            pltpu.VMEM_SHARED((AGG_ROWS, DH), jnp.float32),
            pltpu.SemaphoreType.DMA,
        ],
        compiler_params=pltpu.CompilerParams(use_tc_tiling_on_sc=False),
    )
    def agg_kernel(xs_hbm, src_hbm, dst_hbm, z_hbm, out_hbm,
                   src_v, dst_v, rows_v, agg_sh, gsem):
        cid = lax.axis_index("c")
        sid = lax.axis_index("s")
        x_half = xs_hbm.at[cid]

        pltpu.sync_copy(src_hbm.at[sid], src_v)
        pltpu.sync_copy(dst_hbm.at[sid], dst_v)

        zbase = sid * (AGG_ROWS // NS)
        for j in range(AGG_ROWS // NS // ZROWS):
            pltpu.sync_copy(z_hbm, agg_sh.at[pl.ds(zbase + j * ZROWS, ZROWS)])
        plsc.subcore_barrier()

        for b in range(NBUF):
            pltpu.async_copy(x_half.at[src_v.at[b]], rows_v.at[b], gsem)

        def step(i, carry):
            for b in range(NBUF):
                c = i * NBUF + b
                pltpu.make_async_copy(
                    x_half.at[src_v.at[c]], rows_v.at[b], gsem).wait()
                pltpu.sync_copy(rows_v.at[b], agg_sh.at[dst_v.at[c]], add=True)

                @pl.when(c + NBUF < CHUNKS)
                def _():
                    pltpu.async_copy(
                        x_half.at[src_v.at[c + NBUF]], rows_v.at[b], gsem)
            return carry

        lax.fori_loop(0, CHUNKS // NBUF, step, 0)
        plsc.subcore_barrier()

        obase = sid * OUT_PER_TILE
        pltpu.sync_copy(agg_sh.at[pl.ds(obase, OUT_PER_TILE)],
                        out_hbm.at[cid, pl.ds(obase, OUT_PER_TILE)])

    return agg_kernel(xs, src3, dst3, zeros_blk)


def _tc_mlp(xs, agg, Wa, ba, Wb, bb, relu_out, emit_halves):
    BN = 2000
    grid = (N // BN,)

    def body(xs_ref, agg_ref, wa_ref, ba_ref, wb_ref, bb_ref, o_ref):
        rst = jnp.concatenate(
            [xs_ref[0] + agg_ref[0], xs_ref[1] + agg_ref[1]], axis=1)
        hid = jnp.dot(rst, wa_ref[...], preferred_element_type=jnp.float32)
        hid = jnp.maximum(hid + ba_ref[...], 0.0)
        out = jnp.dot(hid, wb_ref[...], preferred_element_type=jnp.float32)
        out = out + bb_ref[...]
        if relu_out:
            out = jnp.maximum(out, 0.0)
        if emit_halves:
            o_ref[0] = out[:, :DH]
            o_ref[1] = out[:, DH:]
        else:
            o_ref[...] = out

    halves_spec = pl.BlockSpec((NC, BN, DH), lambda i: (0, i, 0))
    full_spec = pl.BlockSpec((D, D), lambda i: (0, 0))
    vec_spec = pl.BlockSpec((1, D), lambda i: (0, 0))
    if emit_halves:
        out_spec = halves_spec
        out_shape = jax.ShapeDtypeStruct((NC, N, DH), jnp.float32)
    else:
        out_spec = pl.BlockSpec((BN, D), lambda i: (i, 0))
        out_shape = jax.ShapeDtypeStruct((N, D), jnp.float32)
    return pl.pallas_call(
        body,
        grid=grid,
        in_specs=[halves_spec, halves_spec,
                  full_spec, vec_spec, full_spec, vec_spec],
        out_specs=out_spec,
        out_shape=out_shape,
    )(xs, agg[:, :N], Wa, ba.reshape(1, D), Wb, bb.reshape(1, D))


def kernel(h, edge_index, W0a, b0a, W0b, b0b, W1a, b1a, W1b, b1b):
    xs0 = h.reshape(NC, DH, N).transpose(0, 2, 1)

    pad = E_PAD - E
    src = jnp.concatenate([edge_index[0], jnp.zeros((pad,), jnp.int32)])
    dst = jnp.concatenate([edge_index[1], jnp.full((pad,), N, jnp.int32)])
    src3 = src.reshape(NS, CHUNKS, CHUNK)
    dst3 = dst.reshape(NS, CHUNKS, CHUNK)
    zeros_blk = jnp.zeros((ZROWS, DH), jnp.float32)

    p0 = _sc_aggregate(xs0, src3, dst3, zeros_blk)
    xs1 = _tc_mlp(xs0, p0, W0a, b0a, W0b, b0b, relu_out=True, emit_halves=True)
    p1 = _sc_aggregate(xs1, src3, dst3, zeros_blk)
    out = _tc_mlp(xs1, p1, W1a, b1a, W1b, b1b, relu_out=False,
                  emit_halves=False)
    return out.T
